# baseline (speedup 1.0000x reference)
import functools

import jax
import jax.numpy as jnp
from jax import lax
from jax.experimental import pallas as pl
from jax.experimental.pallas import tpu as pltpu
from jax.experimental.pallas import tpu_sc as plsc

N_NODES = 10000
N_EDGES = 320000
NC = 2
NS = 16
NW = NC * NS
EPW = N_EDGES // NW
CH = 125
NCH = EPW // CH
NPAD = 10240
RPT = NPAD // NS
ZR = 80

_EPS = 1e-5


def _make_segsum(width, NB):
    mesh = plsc.VectorSubcoreMesh(core_axis_name="c", subcore_axis_name="s")

    @functools.partial(
        pl.kernel,
        mesh=mesh,
        out_type=jax.ShapeDtypeStruct((NC, NPAD, width), jnp.float32),
        scratch_types=[
            pltpu.VMEM((NCH, CH), jnp.int32),
            pltpu.VMEM((NCH, CH), jnp.int32),
            pltpu.VMEM((NB, CH, width), jnp.float32),
            pltpu.VMEM_SHARED((NPAD, width), jnp.float32),
            pltpu.SemaphoreType.DMA((NB,)),
            pltpu.SemaphoreType.DMA((NB,)),
        ],
        compiler_params=pltpu.CompilerParams(use_tc_tiling_on_sc=False),
    )
    def seg(y_hbm, ei_hbm, z_hbm, out_hbm,
            src_v, dst_v, rows_v, acc_sh, gsem, ssem):
        c = lax.axis_index("c")
        s = lax.axis_index("s")
        w = c * NS + s
        pltpu.sync_copy(ei_hbm.at[0, w], src_v)
        pltpu.sync_copy(ei_hbm.at[1, w], dst_v)
        for z in range(RPT // ZR):
            pltpu.sync_copy(z_hbm, acc_sh.at[pl.ds(s * RPT + z * ZR, ZR)])
        for k in range(NB - 1):
            pltpu.async_copy(y_hbm.at[src_v.at[k]], rows_v.at[k], gsem.at[k])
        plsc.subcore_barrier()

        def _gwait(k):
            pltpu.make_async_copy(y_hbm.at[src_v.at[0]], rows_v.at[k],
                                  gsem.at[k]).wait()

        def _swait(k):
            pltpu.make_async_copy(y_hbm.at[src_v.at[0]], rows_v.at[k],
                                  ssem.at[k]).wait()

        def body(i, carry):
            for k in range(NB):
                j = i * NB + k
                jn = j + NB - 1
                kn = (k + NB - 1) % NB

                @pl.when(jnp.logical_and(jn >= NB, jn < NCH))
                def _():
                    _swait(kn)

                @pl.when(jn < NCH)
                def _():
                    pltpu.async_copy(y_hbm.at[src_v.at[jn]], rows_v.at[kn],
                                     gsem.at[kn])

                _gwait(k)
                pltpu.async_copy(rows_v.at[k], acc_sh.at[dst_v.at[j]],
                                 ssem.at[k], add=True)
            return carry

        lax.fori_loop(0, NCH // NB, body, 0)
        for k in range(NB):
            _swait(k)
        plsc.subcore_barrier()
        pltpu.sync_copy(acc_sh.at[pl.ds(s * RPT, RPT)],
                        out_hbm.at[c, pl.ds(s * RPT, RPT)])

    return seg


@functools.lru_cache(maxsize=None)
def _get_segsum(width, nb):
    return _make_segsum(width, nb)


def _pre_body(x_ref, w_ref, o_ref):
    y = lax.dot_general(x_ref[...], w_ref[...], (((1,), (1,)), ((), ())),
                        preferred_element_type=jnp.float32)
    col = lax.broadcasted_iota(jnp.int32, y.shape, 1)
    o_ref[...] = y + jnp.where(col == 64, 1.0, 0.0)


def _mid_body(p_ref, x_ref, w0r_ref, b0l_ref, g0_ref, be0_ref, w1l_ref,
              h_ref, y1_ref, cnt_ref):
    S = (p_ref[0] + p_ref[1])[0:N_NODES]
    cnt = jnp.maximum(S[:, 64:65], 1.0)
    h = (S[:, 0:64] / cnt + b0l_ref[...] +
         lax.dot_general(x_ref[...], w0r_ref[...], (((1,), (1,)), ((), ())),
                         preferred_element_type=jnp.float32))
    mu = jnp.mean(h, axis=0, keepdims=True)
    var = jnp.mean((h - mu) * (h - mu), axis=0, keepdims=True)
    h = g0_ref[...] * (h - mu) / jnp.sqrt(var + _EPS) + be0_ref[...]
    h = jnp.maximum(h, 0.0)
    h_ref[...] = h
    y1_ref[...] = lax.dot_general(h, w1l_ref[...], (((1,), (1,)), ((), ())),
                                  preferred_element_type=jnp.float32)
    cnt_ref[...] = cnt


def _fin_body(q_ref, h_ref, w1r_ref, b1l_ref, g1_ref, be1_ref, cnt_ref,
              o_ref):
    S = (q_ref[0] + q_ref[1])[0:N_NODES]
    t = (S / cnt_ref[...] + b1l_ref[...] +
         lax.dot_general(h_ref[...], w1r_ref[...], (((1,), (1,)), ((), ())),
                         preferred_element_type=jnp.float32))
    mu = jnp.mean(t, axis=0, keepdims=True)
    var = jnp.mean((t - mu) * (t - mu), axis=0, keepdims=True)
    o_ref[...] = g1_ref[...] * (t - mu) / jnp.sqrt(var + _EPS) + be1_ref[...]


def kernel(x, edge_index, W0l, b0l, W0r, gamma0, beta0,
           W1l, b1l, W1r, gamma1, beta1):
    f32 = jnp.float32
    ei = edge_index.astype(jnp.int32).reshape(2, NW, NCH, CH)
    w0l_pad = jnp.concatenate([W0l, jnp.zeros((16, 128), f32)], axis=0)
    z80 = jnp.zeros((ZR, 80), f32)
    z32 = jnp.zeros((ZR, 32), f32)

    y0 = pl.pallas_call(
        _pre_body,
        out_shape=jax.ShapeDtypeStruct((N_NODES, 80), f32),
    )(x, w0l_pad)

    p0 = _get_segsum(80, 5)(y0, ei, z80)

    h, y1, cnt = pl.pallas_call(
        _mid_body,
        out_shape=(
            jax.ShapeDtypeStruct((N_NODES, 64), f32),
            jax.ShapeDtypeStruct((N_NODES, 32), f32),
            jax.ShapeDtypeStruct((N_NODES, 1), f32),
        ),
    )(p0, x, W0r, b0l.reshape(1, 64), gamma0.reshape(1, 64),
      beta0.reshape(1, 64), W1l)

    p1 = _get_segsum(32, 8)(y1, ei, z32)

    out = pl.pallas_call(
        _fin_body,
        out_shape=jax.ShapeDtypeStruct((N_NODES, 32), f32),
    )(p1, h, W1r, b1l.reshape(1, 32), gamma1.reshape(1, 32),
      beta1.reshape(1, 32), cnt)
    return out


# speedup vs baseline: 1.1963x; 1.1963x over previous
import functools

import jax
import jax.numpy as jnp
from jax import lax
from jax.experimental import pallas as pl
from jax.experimental.pallas import tpu as pltpu
from jax.experimental.pallas import tpu_sc as plsc

N_NODES = 10000
N_EDGES = 320000
NC = 2
NS = 16
NW = NC * NS
EPW = N_EDGES // NW
CH = 125
NCH = EPW // CH
NPAD = 10240
RPT = NPAD // NS

_EPS = 1e-5


def _make_segsum(width, NB):
    mesh = plsc.VectorSubcoreMesh(core_axis_name="c", subcore_axis_name="s")

    @functools.partial(
        pl.kernel,
        mesh=mesh,
        out_type=jax.ShapeDtypeStruct((NC, NPAD, 128), jnp.float32),
        scratch_types=[
            pltpu.VMEM((NCH, CH), jnp.int32),
            pltpu.VMEM((NCH, CH), jnp.int32),
            pltpu.VMEM((NB, CH, width), jnp.float32),
            pltpu.VMEM_SHARED((NPAD, width), jnp.float32),
            pltpu.SemaphoreType.DMA((NB,)),
            pltpu.SemaphoreType.DMA((NB,)),
        ],
        compiler_params=pltpu.CompilerParams(use_tc_tiling_on_sc=False),
    )
    def seg(y_hbm, ei_hbm, z_hbm, out_hbm,
            src_v, dst_v, rows_v, acc_sh, gsem, ssem):
        c = lax.axis_index("c")
        s = lax.axis_index("s")
        w = c * NS + s
        pltpu.sync_copy(ei_hbm.at[0, w], src_v)
        pltpu.sync_copy(ei_hbm.at[1, w], dst_v)
        pltpu.sync_copy(z_hbm, acc_sh.at[pl.ds(s * RPT, RPT)])
        for k in range(NB - 1):
            pltpu.async_copy(y_hbm.at[src_v.at[k]], rows_v.at[k], gsem.at[k])
        plsc.subcore_barrier()

        def _gwait(k):
            pltpu.make_async_copy(y_hbm.at[src_v.at[0]], rows_v.at[k],
                                  gsem.at[k]).wait()

        def _swait(k):
            pltpu.make_async_copy(y_hbm.at[src_v.at[0]], rows_v.at[k],
                                  ssem.at[k]).wait()

        def body(i, carry):
            for k in range(NB):
                j = i * NB + k
                jn = j + NB - 1
                kn = (k + NB - 1) % NB

                @pl.when(jnp.logical_and(jn >= NB, jn < NCH))
                def _():
                    _swait(kn)

                @pl.when(jn < NCH)
                def _():
                    pltpu.async_copy(y_hbm.at[src_v.at[jn]], rows_v.at[kn],
                                     gsem.at[kn])

                _gwait(k)
                pltpu.async_copy(rows_v.at[k], acc_sh.at[dst_v.at[j]],
                                 ssem.at[k], add=True)
            return carry

        lax.fori_loop(0, NCH // NB, body, 0)
        for k in range(NB):
            _swait(k)
        plsc.subcore_barrier()
        pltpu.sync_copy(acc_sh.at[pl.ds(s * RPT, RPT)],
                        out_hbm.at[c, pl.ds(s * RPT, RPT), pl.ds(0, width)])

    return seg


@functools.lru_cache(maxsize=None)
def _get_segsum(width, nb):
    return _make_segsum(width, nb)


def _pre_body(x_ref, w_ref, o_ref):
    y = lax.dot_general(x_ref[...], w_ref[...], (((1,), (1,)), ((), ())),
                        preferred_element_type=jnp.float32)
    col = lax.broadcasted_iota(jnp.int32, y.shape, 1)
    o_ref[...] = y + jnp.where(col == 64, 1.0, 0.0)


def _mid_body(p_ref, x_ref, w0r_ref, b0l_ref, g0_ref, be0_ref, w1l_ref,
              h_ref, y1_ref, cnt_ref):
    S = (p_ref[0] + p_ref[1])[0:N_NODES]
    cnt = jnp.maximum(S[:, 64:65], 1.0)
    h = (S[:, 0:64] / cnt + b0l_ref[...] +
         lax.dot_general(x_ref[...], w0r_ref[...], (((1,), (1,)), ((), ())),
                         preferred_element_type=jnp.float32))
    mu = jnp.mean(h, axis=0, keepdims=True)
    var = jnp.mean((h - mu) * (h - mu), axis=0, keepdims=True)
    h = g0_ref[...] * (h - mu) / jnp.sqrt(var + _EPS) + be0_ref[...]
    h = jnp.maximum(h, 0.0)
    h_ref[...] = h
    y1_ref[...] = lax.dot_general(h, w1l_ref[...], (((1,), (1,)), ((), ())),
                                  preferred_element_type=jnp.float32)
    cnt_ref[...] = cnt


def _fin_body(q_ref, h_ref, w1r_ref, b1l_ref, g1_ref, be1_ref, cnt_ref,
              o_ref):
    S = (q_ref[0] + q_ref[1])[0:N_NODES, 0:32]
    t = (S / cnt_ref[...] + b1l_ref[...] +
         lax.dot_general(h_ref[...], w1r_ref[...], (((1,), (1,)), ((), ())),
                         preferred_element_type=jnp.float32))
    mu = jnp.mean(t, axis=0, keepdims=True)
    var = jnp.mean((t - mu) * (t - mu), axis=0, keepdims=True)
    o_ref[...] = g1_ref[...] * (t - mu) / jnp.sqrt(var + _EPS) + be1_ref[...]


def kernel(x, edge_index, W0l, b0l, W0r, gamma0, beta0,
           W1l, b1l, W1r, gamma1, beta1):
    f32 = jnp.float32
    ei = edge_index.astype(jnp.int32).reshape(2, NW, NCH, CH)
    w0l_pad = jnp.concatenate([W0l, jnp.zeros((16, 128), f32)], axis=0)
    z80 = jnp.zeros((RPT, 80), f32)
    z32 = jnp.zeros((RPT, 32), f32)

    y0 = pl.pallas_call(
        _pre_body,
        out_shape=jax.ShapeDtypeStruct((N_NODES, 80), f32),
    )(x, w0l_pad)

    p0 = _get_segsum(80, 5)(y0, ei, z80)

    h, y1, cnt = pl.pallas_call(
        _mid_body,
        out_shape=(
            jax.ShapeDtypeStruct((N_NODES, 64), f32),
            jax.ShapeDtypeStruct((N_NODES, 32), f32),
            jax.ShapeDtypeStruct((N_NODES, 1), f32),
        ),
    )(p0, x, W0r, b0l.reshape(1, 64), gamma0.reshape(1, 64),
      beta0.reshape(1, 64), W1l)

    p1 = _get_segsum(32, 8)(y1, ei, z32)

    out = pl.pallas_call(
        _fin_body,
        out_shape=jax.ShapeDtypeStruct((N_NODES, 32), f32),
    )(p1, h, W1r, b1l.reshape(1, 32), gamma1.reshape(1, 32),
      beta1.reshape(1, 32), cnt)
    return out
